# Initial kernel scaffold; baseline (speedup 1.0000x reference)
import jax
import jax.numpy as jnp
from jax.experimental import pallas as pl


def kernel(x, edge_index, edge_attr, batch, emb, Wpost, bpost, gamma, beta, W1, b1, W2, b2, W3, b3):
    raise NotImplementedError("write your pallas kernel here")


# speedup vs baseline: 4.2512x; 4.2512x over previous
import functools
import numpy as np
import jax
import jax.numpy as jnp
from jax import lax
from jax.experimental import pallas as pl
from jax.experimental.pallas import tpu as pltpu
from jax.experimental.pallas import tpu_sc as plsc

N = 10000
E = 320000
D = 70
DE = 16
G = 400
NUM_LAYERS = 4
_DEG_HIST = np.array([0, 100, 500, 1500, 2500, 2000, 1500, 1000, 500, 250, 100, 50], dtype=np.float64)
DELTA = float((_DEG_HIST * np.log(np.arange(len(_DEG_HIST)) + 1.0)).sum() / _DEG_HIST.sum())

NW = 32
NPW = 320
NPAD = NW * NPW
FP = 80
CHP = 2000
NCHP = E // CHP
FLUSH = CHP + 16
EC = 324608
CH = 128
CH2 = 512
NEG = -3.0e38
POS = 3.0e38

_SC_PARAMS = pltpu.CompilerParams(use_tc_tiling_on_sc=False, needs_layout_passes=False)
_mesh = plsc.VectorSubcoreMesh(core_axis_name="c", subcore_axis_name="s")


def _wid():
    return lax.axis_index("s") * 2 + lax.axis_index("c")


def _extract(ref, i):
    return ref[pl.ds(i, 16)][0]


def _partition_body(src_hbm, dst_hbm, ea_hbm, x_hbm, emb_hbm,
                    srcp, dstp, eidp, cnts, deg_o, eas_o, eaq_o, eam_o, ean_o, h0_o,
                    sbuf, dbuf, cbs, cbd, cbe, degacc, aas, aaq, aam, aan,
                    e2, d2, earows, xbuf, hrows, cvec, sem):
    wid = _wid()
    lo = pl.multiple_of(wid * NPW, 8)
    wbase = pl.multiple_of(wid * EC, 64)
    iota = lax.broadcasted_iota(jnp.int32, (16,), 0)
    lane0 = jnp.where(iota == 0, 1.0, 0.0)
    zs16 = jnp.zeros((16,), jnp.int32)
    dumd = zs16 + NPW

    pltpu.sync_copy(x_hbm.at[pl.ds(lo, NPW)], xbuf)
    pltpu.async_copy(emb_hbm.at[xbuf], hrows, sem).wait()
    pltpu.sync_copy(hrows, h0_o.at[pl.ds(lo, NPW)])

    def chunk(c, off):
        cb = pl.multiple_of(c * CHP, 16)
        pltpu.sync_copy(src_hbm.at[pl.ds(cb, CHP)], sbuf)
        pltpu.sync_copy(dst_hbm.at[pl.ds(cb, CHP)], dbuf)

        def vec(v, k):
            dv = dbuf[pl.ds(v * 16, 16)]
            sv = sbuf[pl.ds(v * 16, 16)]
            dl = dv - lo
            m = (dl >= 0) & (dl < NPW)
            ev = zs16 + cb + v * 16 + iota
            inc = jnp.where(m, 1, 0)
            csum = plsc.cumsum(inc)
            pos = jnp.where(m, k + csum - inc, FLUSH + 8)
            plsc.store_scatter(cbs, [pos], sv)
            plsc.store_scatter(cbd, [pos], dl)
            plsc.store_scatter(cbe, [pos], ev)
            return k + csum[15]

        k = lax.fori_loop(0, CHP // 16, vec, 0)
        kio = k + iota
        plsc.store_scatter(cbs, [kio], zs16)
        plsc.store_scatter(cbd, [kio], dumd)
        plsc.store_scatter(cbe, [kio], zs16)
        fo = pl.multiple_of(wbase + off, 16)
        pltpu.sync_copy(cbs.at[pl.ds(0, FLUSH)], srcp.at[pl.ds(fo, FLUSH)])
        pltpu.sync_copy(cbd.at[pl.ds(0, FLUSH)], dstp.at[pl.ds(fo, FLUSH)])
        pltpu.sync_copy(cbe.at[pl.ds(0, FLUSH)], eidp.at[pl.ds(fo, FLUSH)])
        kpad = ((k + 15) // 16) * 16
        return off + kpad

    cnt = lax.fori_loop(0, NCHP, chunk, 0)
    cvec[pl.ds(0, 16)] = zs16 + cnt
    pltpu.sync_copy(cvec, cnts.at[pl.ds(pl.multiple_of(wid * 16, 16), 16)])

    def zinit(i, _):
        degacc[i, pl.ds(0, 16)] = jnp.zeros((16,), jnp.float32)
        aas[i, pl.ds(0, 16)] = jnp.zeros((16,), jnp.float32)
        aaq[i, pl.ds(0, 16)] = jnp.zeros((16,), jnp.float32)
        aam[i, pl.ds(0, 16)] = jnp.full((16,), NEG, jnp.float32)
        aan[i, pl.ds(0, 16)] = jnp.full((16,), POS, jnp.float32)
        return 0

    lax.fori_loop(0, NPW + 1, zinit, 0)

    def chunk2(c, _):
        cb = pl.multiple_of(wbase + c * CH2, 16)
        pltpu.sync_copy(dstp.at[pl.ds(cb, CH2)], d2)
        pltpu.sync_copy(eidp.at[pl.ds(cb, CH2)], e2)

        def clampv(v, _):
            ev = e2[pl.ds(v * 16, 16)]
            e2[pl.ds(v * 16, 16)] = jnp.minimum(jnp.maximum(ev, 0), E - 1)
            return 0

        lax.fori_loop(0, CH2 // 16, clampv, 0)
        pltpu.async_copy(ea_hbm.at[e2], earows, sem).wait()
        ng = jnp.minimum(CH2, cnt - c * CH2) // 16

        def grp(g, _):
            dv = d2[pl.ds(g * 16, 16)]
            for j in range(16):
                d = dv[j]
                r = earows[g * 16 + j, pl.ds(0, 16)]
                degacc[d, pl.ds(0, 16)] = degacc[d, pl.ds(0, 16)] + lane0
                aas[d, pl.ds(0, 16)] = aas[d, pl.ds(0, 16)] + r
                aaq[d, pl.ds(0, 16)] = aaq[d, pl.ds(0, 16)] + r * r
                aam[d, pl.ds(0, 16)] = jnp.maximum(aam[d, pl.ds(0, 16)], r)
                aan[d, pl.ds(0, 16)] = jnp.minimum(aan[d, pl.ds(0, 16)], r)
            return 0

        lax.fori_loop(0, ng, grp, 0)
        return 0

    nch2 = (cnt + CH2 - 1) // CH2
    lax.fori_loop(0, nch2, chunk2, 0)

    pltpu.sync_copy(degacc.at[pl.ds(0, NPW)], deg_o.at[pl.ds(lo, NPW)])
    pltpu.sync_copy(aas.at[pl.ds(0, NPW)], eas_o.at[pl.ds(lo, NPW)])
    pltpu.sync_copy(aaq.at[pl.ds(0, NPW)], eaq_o.at[pl.ds(lo, NPW)])
    pltpu.sync_copy(aam.at[pl.ds(0, NPW)], eam_o.at[pl.ds(lo, NPW)])
    pltpu.sync_copy(aan.at[pl.ds(0, NPW)], ean_o.at[pl.ds(lo, NPW)])


def _partition(src, dst, ea, x, emb):
    f32 = jnp.float32
    f = pl.kernel(
        _partition_body,
        out_type=(
            jax.ShapeDtypeStruct((NW * EC,), jnp.int32),
            jax.ShapeDtypeStruct((NW * EC,), jnp.int32),
            jax.ShapeDtypeStruct((NW * EC,), jnp.int32),
            jax.ShapeDtypeStruct((NW * 16,), jnp.int32),
            jax.ShapeDtypeStruct((NPAD, 16), f32),
            jax.ShapeDtypeStruct((NPAD, 16), f32),
            jax.ShapeDtypeStruct((NPAD, 16), f32),
            jax.ShapeDtypeStruct((NPAD, 16), f32),
            jax.ShapeDtypeStruct((NPAD, 16), f32),
            jax.ShapeDtypeStruct((NPAD, FP), f32),
        ),
        mesh=_mesh,
        compiler_params=_SC_PARAMS,
        scratch_types=[
            pltpu.VMEM((CHP,), jnp.int32),
            pltpu.VMEM((CHP,), jnp.int32),
            pltpu.VMEM((FLUSH + 16,), jnp.int32),
            pltpu.VMEM((FLUSH + 16,), jnp.int32),
            pltpu.VMEM((FLUSH + 16,), jnp.int32),
            pltpu.VMEM((NPW + 1, 16), f32),
            pltpu.VMEM((NPW + 1, 16), f32),
            pltpu.VMEM((NPW + 1, 16), f32),
            pltpu.VMEM((NPW + 1, 16), f32),
            pltpu.VMEM((NPW + 1, 16), f32),
            pltpu.VMEM((CH2,), jnp.int32),
            pltpu.VMEM((CH2,), jnp.int32),
            pltpu.VMEM((CH2, 16), f32),
            pltpu.VMEM((NPW,), jnp.int32),
            pltpu.VMEM((NPW, FP), f32),
            pltpu.VMEM((16,), jnp.int32),
            pltpu.SemaphoreType.DMA,
        ],
    )
    return f(src, dst, ea, x, emb)


def _agg_body(h_hbm, srcp, dstp, cnts, ssum_o, ssq_o, smx_o, smn_o,
              idxb, dstb, rows, asum, asq, amx, amn, cvec, sem):
    wid = _wid()
    lo = pl.multiple_of(wid * NPW, 8)
    wbase = pl.multiple_of(wid * EC, 64)
    pltpu.sync_copy(cnts.at[pl.ds(pl.multiple_of(wid * 16, 16), 16)], cvec)
    cnt = cvec[pl.ds(0, 16)][0]

    def zinit(i, _):
        for t in range(FP // 16):
            sl = pl.ds(t * 16, 16)
            asum[i, sl] = jnp.zeros((16,), jnp.float32)
            asq[i, sl] = jnp.zeros((16,), jnp.float32)
            amx[i, sl] = jnp.full((16,), NEG, jnp.float32)
            amn[i, sl] = jnp.full((16,), POS, jnp.float32)
        return 0

    lax.fori_loop(0, NPW + 1, zinit, 0)

    def chunk(c, _):
        cb = pl.multiple_of(wbase + c * CH, 16)
        pltpu.sync_copy(srcp.at[pl.ds(cb, CH)], idxb)
        pltpu.sync_copy(dstp.at[pl.ds(cb, CH)], dstb)

        def clampv(v, _):
            iv = idxb[pl.ds(v * 16, 16)]
            idxb[pl.ds(v * 16, 16)] = jnp.minimum(jnp.maximum(iv, 0), NPAD - 1)
            return 0

        lax.fori_loop(0, CH // 16, clampv, 0)
        pltpu.async_copy(h_hbm.at[idxb], rows, sem).wait()
        ng = jnp.minimum(CH, cnt - c * CH) // 16

        def grp(g, _):
            dv = dstb[pl.ds(g * 16, 16)]
            for j in range(16):
                d = dv[j]
                i = g * 16 + j
                for t in range(FP // 16):
                    sl = pl.ds(t * 16, 16)
                    r = rows[i, sl]
                    asum[d, sl] = asum[d, sl] + r
                    asq[d, sl] = asq[d, sl] + r * r
                    amx[d, sl] = jnp.maximum(amx[d, sl], r)
                    amn[d, sl] = jnp.minimum(amn[d, sl], r)
            return 0

        lax.fori_loop(0, ng, grp, 0)
        return 0

    nch = (cnt + CH - 1) // CH
    lax.fori_loop(0, nch, chunk, 0)

    pltpu.sync_copy(asum.at[pl.ds(0, NPW)], ssum_o.at[pl.ds(lo, NPW)])
    pltpu.sync_copy(asq.at[pl.ds(0, NPW)], ssq_o.at[pl.ds(lo, NPW)])
    pltpu.sync_copy(amx.at[pl.ds(0, NPW)], smx_o.at[pl.ds(lo, NPW)])
    pltpu.sync_copy(amn.at[pl.ds(0, NPW)], smn_o.at[pl.ds(lo, NPW)])


def _aggregate(h, srcp, dstp, cnts):
    f32 = jnp.float32
    sds = jax.ShapeDtypeStruct((NPAD, FP), f32)
    f = pl.kernel(
        _agg_body,
        out_type=(sds, sds, sds, sds),
        mesh=_mesh,
        compiler_params=_SC_PARAMS,
        scratch_types=[
            pltpu.VMEM((CH,), jnp.int32),
            pltpu.VMEM((CH,), jnp.int32),
            pltpu.VMEM((CH, FP), f32),
            pltpu.VMEM((NPW + 1, FP), f32),
            pltpu.VMEM((NPW + 1, FP), f32),
            pltpu.VMEM((NPW + 1, FP), f32),
            pltpu.VMEM((NPW + 1, FP), f32),
            pltpu.VMEM((16,), jnp.int32),
            pltpu.SemaphoreType.DMA,
        ],
    )
    return f(h, srcp, dstp, cnts)


NB = 256
NGRID = NPAD // NB
KP = 176
SQ_EPS = float(np.sqrt(1e-5))


def _dense_body(h_r, ssum_r, ssq_r, smx_r, smn_r, deg_r,
                eas_r, eaq_r, eam_r, ean_r, w_r, b_r, z_r, st_r):
    i = pl.program_id(0)
    f32 = jnp.float32
    deg0 = deg_r[:, 0:1]
    has = deg0 > 0.0
    degc = jnp.maximum(deg0, 1.0)
    logd = jnp.log(deg0 + 1.0)
    amp = logd / DELTA
    att = jnp.where(has, DELTA / jnp.maximum(logd, 1e-6), 1.0)

    h = h_r[...]
    b1 = jnp.where(has, h, 0.0)
    mean2 = ssum_r[...] / degc
    std2 = jnp.sqrt(jnp.maximum(ssq_r[...] / degc - mean2 * mean2, 0.0) + 1e-5)
    mx2 = jnp.where(has, smx_r[...], 0.0)
    mn2 = jnp.where(has, smn_r[...], 0.0)
    mean3 = eas_r[...] / degc
    std3 = jnp.sqrt(jnp.maximum(eaq_r[...] / degc - mean3 * mean3, 0.0) + 1e-5)
    mx3 = jnp.where(has, eam_r[...], 0.0)
    mn3 = jnp.where(has, ean_r[...], 0.0)
    std1 = jnp.full(b1.shape, SQ_EPS, f32)

    pieces = [
        jnp.concatenate([b1, mean2, mean3], axis=1),
        jnp.concatenate([b1, mn2, mn3], axis=1),
        jnp.concatenate([b1, mx2, mx3], axis=1),
        jnp.concatenate([std1, std2, std3], axis=1),
    ]
    accs = []
    for s in range(3):
        acc = jnp.zeros((NB, 128), f32)
        for a in range(4):
            acc = acc + jax.lax.dot_general(
                pieces[a], w_r[s * 4 + a],
                (((1,), (0,)), ((), ())), preferred_element_type=f32)
        accs.append(acc)
    z = accs[0] + amp * accs[1] + att * accs[2] + b_r[0:1, :]
    zc = z[:, 0:FP]
    z_r[...] = zc

    rows = i * NB + lax.broadcasted_iota(jnp.int32, (NB, 1), 0)
    zm = jnp.where(rows < N, zc, 0.0)
    s1 = zm.reshape(NB // 8, 8, FP).sum(axis=0)
    s2 = (zm * zm).reshape(NB // 8, 8, FP).sum(axis=0)

    @pl.when(i == 0)
    def _():
        st_r[...] = jnp.zeros((16, FP), f32)

    st_r[0:8, :] += s1
    st_r[8:16, :] += s2


def _dense(h, ssum, ssq, smx, smn, deg, eas, eaq, eam, ean, wl, bl):
    f32 = jnp.float32
    bs_n = lambda: pl.BlockSpec((NB, FP), lambda i: (i, 0))
    bs_16 = lambda: pl.BlockSpec((NB, 16), lambda i: (i, 0))
    return pl.pallas_call(
        _dense_body,
        grid=(NGRID,),
        in_specs=[
            bs_n(), bs_n(), bs_n(), bs_n(), bs_n(), bs_16(),
            bs_16(), bs_16(), bs_16(), bs_16(),
            pl.BlockSpec((12, KP, 128), lambda i: (0, 0, 0)),
            pl.BlockSpec((8, 128), lambda i: (0, 0)),
        ],
        out_specs=[
            pl.BlockSpec((NB, FP), lambda i: (i, 0)),
            pl.BlockSpec((16, FP), lambda i: (0, 0)),
        ],
        out_shape=[
            jax.ShapeDtypeStruct((NPAD, FP), f32),
            jax.ShapeDtypeStruct((16, FP), f32),
        ],
    )(h, ssum, ssq, smx, smn, deg, eas, eaq, eam, ean, wl, bl)


def _bnres_body(z_r, h_r, st_r, gb_r, o_r):
    mu = st_r[0:8, :].sum(axis=0, keepdims=True) / float(N)
    msq = st_r[8:16, :].sum(axis=0, keepdims=True) / float(N)
    var = msq - mu * mu
    gamma = gb_r[0:1, :]
    beta = gb_r[8:9, :]
    z = z_r[...]
    zn = gamma * (z - mu) / jnp.sqrt(var + 1e-5) + beta
    o_r[...] = jnp.maximum(zn, 0.0) + h_r[...]


def _bnres(z, h, st, gb):
    return pl.pallas_call(
        _bnres_body,
        grid=(NGRID,),
        in_specs=[
            pl.BlockSpec((NB, FP), lambda i: (i, 0)),
            pl.BlockSpec((NB, FP), lambda i: (i, 0)),
            pl.BlockSpec((16, FP), lambda i: (0, 0)),
            pl.BlockSpec((16, FP), lambda i: (0, 0)),
        ],
        out_specs=pl.BlockSpec((NB, FP), lambda i: (i, 0)),
        out_shape=jax.ShapeDtypeStruct((NPAD, FP), jnp.float32),
    )(z, h, st, gb)


def _pool_body(h_r, bf_r, o_r):
    i = pl.program_id(0)
    hb = jnp.concatenate(
        [h_r[...], jnp.ones((NB, 16), jnp.float32)], axis=1)
    cols = lax.broadcasted_iota(jnp.int32, (NB, G), 1).astype(jnp.float32)
    oh = jnp.where(bf_r[...] == cols, 1.0, 0.0)
    contrib = jax.lax.dot_general(
        oh, hb, (((0,), (0,)), ((), ())), preferred_element_type=jnp.float32)

    @pl.when(i == 0)
    def _():
        o_r[...] = jnp.zeros((G, FP + 16), jnp.float32)

    o_r[...] += contrib


def _pool(h, bf):
    return pl.pallas_call(
        _pool_body,
        grid=(NGRID,),
        in_specs=[
            pl.BlockSpec((NB, FP), lambda i: (i, 0)),
            pl.BlockSpec((NB, 1), lambda i: (i, 0)),
        ],
        out_specs=pl.BlockSpec((G, FP + 16), lambda i: (0, 0)),
        out_shape=jax.ShapeDtypeStruct((G, FP + 16), jnp.float32),
    )(h, bf)


def _mlp_body(p_r, w1_r, b1_r, w2_r, b2_r, w3_r, b3_r, o_r):
    f32 = jnp.float32
    cnt = jnp.maximum(p_r[:, FP:FP + 1], 1.0)
    g = p_r[:, 0:D] / cnt
    a1 = jnp.maximum(jax.lax.dot_general(
        g, w1_r[...], (((1,), (0,)), ((), ())), preferred_element_type=f32) + b1_r[0:1, :], 0.0)
    a2 = jnp.maximum(jax.lax.dot_general(
        a1, w2_r[...], (((1,), (0,)), ((), ())), preferred_element_type=f32) + b2_r[0:1, :], 0.0)
    a3 = jax.lax.dot_general(
        a2, w3_r[...], (((1,), (0,)), ((), ())), preferred_element_type=f32) + b3_r[0:1, :]
    o_r[...] = jnp.broadcast_to(a3, (G, 8))


def _mlp(pooled, w1, b1, w2, b2, w3, b3):
    full = lambda *shape: pl.BlockSpec(shape, lambda: tuple(0 for _ in shape))
    return pl.pallas_call(
        _mlp_body,
        in_specs=[
            full(G, FP + 16), full(D, 35), full(1, 35),
            full(35, 17), full(1, 17), full(17, 1), full(1, 1),
        ],
        out_specs=full(G, 8),
        out_shape=jax.ShapeDtypeStruct((G, 8), jnp.float32),
    )(pooled, w1, b1, w2, b2, w3, b3)


def kernel(x, edge_index, edge_attr, batch, emb, Wpost, bpost, gamma, beta,
           W1, b1, W2, b2, W3, b3):
    f32 = jnp.float32
    src = edge_index[0]
    dst = edge_index[1]
    xp = jnp.pad(x, (0, NPAD - N))
    embp = jnp.pad(emb, ((0, 0), (0, FP - D)))

    w4 = Wpost.reshape(NUM_LAYERS, 3, 4, 156, D)
    wl = jnp.zeros((NUM_LAYERS, 12, KP, 128), f32)
    wl = wl.at[:, :, 0:D, 0:D].set(w4[:, :, :, 0:D, :].reshape(NUM_LAYERS, 12, D, D))
    wl = wl.at[:, :, 80:150, 0:D].set(w4[:, :, :, D:2 * D, :].reshape(NUM_LAYERS, 12, D, D))
    wl = wl.at[:, :, 160:176, 0:D].set(w4[:, :, :, 2 * D:156, :].reshape(NUM_LAYERS, 12, DE, D))
    blp = jnp.zeros((NUM_LAYERS, 8, 128), f32).at[:, :, 0:D].set(bpost[:, None, :])
    gbp = jnp.zeros((NUM_LAYERS, 16, FP), f32)
    gbp = gbp.at[:, 0:8, 0:D].set(gamma[:, None, :])
    gbp = gbp.at[:, 8:16, 0:D].set(beta[:, None, :])

    (srcp, dstp, eidp, cnts, deg, eas, eaq, eam, ean, h0) = _partition(
        src, dst, edge_attr, xp, embp)

    h = h0
    for l in range(NUM_LAYERS):
        ssum, ssq, smx, smn = _aggregate(h, srcp, dstp, cnts)
        z, st = _dense(h, ssum, ssq, smx, smn, deg, eas, eaq, eam, ean,
                       wl[l], blp[l])
        h = _bnres(z, h, st, gbp[l])

    bf = jnp.pad(batch, (0, NPAD - N), constant_values=G).astype(f32)[:, None]
    pooled = _pool(h, bf)
    out = _mlp(pooled, W1, b1[None, :], W2, b2[None, :], W3, b3[None, :])
    return out[:, 0:1]
